# baseline (speedup 1.0000x reference)
import functools

import jax
import jax.numpy as jnp
from jax import lax
from jax.experimental import pallas as pl
from jax.experimental.pallas import tpu as pltpu
from jax.experimental.pallas import tpu_sc as plsc

D_MODEL = 4096
N_EXP = 64
TOPK = 8
TOKENS = 8192

NC, NS, L = 2, 16, 16
NW = NC * NS
ROWS_PER_W = TOKENS // NW
GROUPS = ROWS_PER_W // L


def _matmul_kernel(h_ref, w_ref, logits_ref, logits_t_ref):
    logits_t = jax.lax.dot_general(
        w_ref[...], h_ref[...],
        dimension_numbers=(((1,), (1,)), ((), ())),
        preferred_element_type=jnp.float32,
    )
    logits_t_ref[...] = logits_t
    logits_ref[...] = logits_t.T


def _matmul(h, W, token_block=1024):
    tokens = h.shape[0]
    grid = (tokens // token_block,)
    return pl.pallas_call(
        _matmul_kernel,
        grid=grid,
        in_specs=[
            pl.BlockSpec((token_block, D_MODEL), lambda i: (i, 0)),
            pl.BlockSpec((N_EXP, D_MODEL), lambda i: (0, 0)),
        ],
        out_specs=[
            pl.BlockSpec((token_block, N_EXP), lambda i: (i, 0)),
            pl.BlockSpec((N_EXP, token_block), lambda i: (0, i)),
        ],
        out_shape=[
            jax.ShapeDtypeStruct((tokens, N_EXP), jnp.float32),
            jax.ShapeDtypeStruct((N_EXP, tokens), jnp.float32),
        ],
    )(h, W)


def _make_gate():
    mesh = plsc.VectorSubcoreMesh(core_axis_name="c", subcore_axis_name="s")

    @functools.partial(
        pl.kernel,
        mesh=mesh,
        out_type=[
            jax.ShapeDtypeStruct((TOKENS,), jnp.float32),
            jax.ShapeDtypeStruct((TOKENS,), jnp.float32),
            jax.ShapeDtypeStruct((TOKENS,), jnp.float32),
        ],
        scratch_types=[
            pltpu.VMEM((N_EXP, ROWS_PER_W), jnp.float32),
            pltpu.VMEM((ROWS_PER_W,), jnp.float32),
            pltpu.VMEM((ROWS_PER_W,), jnp.float32),
            pltpu.VMEM((ROWS_PER_W,), jnp.float32),
        ],
    )
    def gate(lt_hbm, rmax_hbm, thr_hbm, den_hbm, lt_v, rmax_v, thr_v, den_v):
        wid = lax.axis_index("s") * NC + lax.axis_index("c")
        base = wid * ROWS_PER_W
        pltpu.sync_copy(lt_hbm.at[:, pl.ds(base, ROWS_PER_W)], lt_v)

        neg_inf = jnp.full((L,), -jnp.inf, dtype=jnp.float32)

        def group_body(g, carry):
            off = g * L
            m = [neg_inf] * TOPK
            for e in range(N_EXP):
                v = lt_v[e, pl.ds(off, L)]
                for i in range(TOPK):
                    hi = jnp.maximum(m[i], v)
                    v = jnp.minimum(m[i], v)
                    m[i] = hi
            rowmax, thresh = m[0], m[TOPK - 1]
            denom = jnp.zeros((L,), dtype=jnp.float32)
            for e in range(N_EXP):
                x = lt_v[e, pl.ds(off, L)]
                ex = jnp.exp(x - rowmax)
                denom = denom + jnp.where(x >= thresh, ex, 0.0)
            rmax_v[pl.ds(off, L)] = rowmax
            thr_v[pl.ds(off, L)] = thresh
            den_v[pl.ds(off, L)] = denom
            return carry

        lax.fori_loop(0, GROUPS, group_body, 0)
        pltpu.sync_copy(rmax_v, rmax_hbm.at[pl.ds(base, ROWS_PER_W)])
        pltpu.sync_copy(thr_v, thr_hbm.at[pl.ds(base, ROWS_PER_W)])
        pltpu.sync_copy(den_v, den_hbm.at[pl.ds(base, ROWS_PER_W)])

    return gate


_gate = _make_gate()


def _apply_kernel(logits_ref, rmax_ref, thr_ref, den_ref, mask_ref, probs_ref):
    logits = logits_ref[...]
    rowmax = rmax_ref[...]
    thresh = thr_ref[...]
    denom = den_ref[...]
    mask = logits >= thresh
    mask_ref[...] = mask
    e = jnp.exp(logits - rowmax)
    probs_ref[...] = jnp.where(mask, e / denom, 0.0)


def _apply(logits, rmax, thr, den, token_block=2048):
    tokens = logits.shape[0]
    grid = (tokens // token_block,)
    stats_spec = pl.BlockSpec((token_block, 1), lambda i: (i, 0))
    return pl.pallas_call(
        _apply_kernel,
        grid=grid,
        in_specs=[
            pl.BlockSpec((token_block, N_EXP), lambda i: (i, 0)),
            stats_spec, stats_spec, stats_spec,
        ],
        out_specs=[
            pl.BlockSpec((token_block, N_EXP), lambda i: (i, 0)),
            pl.BlockSpec((token_block, N_EXP), lambda i: (i, 0)),
        ],
        out_shape=[
            jax.ShapeDtypeStruct((tokens, N_EXP), jnp.bool_),
            jax.ShapeDtypeStruct((tokens, N_EXP), jnp.float32),
        ],
    )(logits, rmax.reshape(tokens, 1), thr.reshape(tokens, 1),
      den.reshape(tokens, 1))


@jax.jit
def kernel(h, W):
    logits, logits_t = _matmul(h, W)
    rmax, thr, den = _gate(logits_t)
    mask, probs = _apply(logits, rmax, thr, den)
    return (mask, probs, logits, logits)


# speedup vs baseline: 1.5369x; 1.5369x over previous
import functools

import jax
import jax.numpy as jnp
from jax.experimental import pallas as pl
from jax.experimental.pallas import tpu as pltpu

D_MODEL = 4096
N_EXP = 64
TOPK = 8
HALF = D_MODEL // 2


def _router_kernel(h1_ref, h2_ref, w_ref, mask_ref, probs_ref, logits_ref):
    dn = (((1,), (1,)), ((), ()))
    logits = jax.lax.dot_general(
        h1_ref[...], w_ref[:, :HALF], dimension_numbers=dn,
        preferred_element_type=jnp.float32,
    ) + jax.lax.dot_general(
        h2_ref[...], w_ref[:, HALF:], dimension_numbers=dn,
        preferred_element_type=jnp.float32,
    )
    logits_ref[...] = logits

    tb = logits.shape[0]
    work = logits
    mask = jnp.zeros((tb, N_EXP), dtype=jnp.bool_)
    rowmax = None
    for i in range(TOPK):
        m = jnp.max(work, axis=1, keepdims=True)
        if i == 0:
            rowmax = m
        sel = work == m
        mask = jnp.logical_or(mask, sel)
        work = jnp.where(sel, -jnp.inf, work)
    mask_ref[...] = mask

    e = jnp.exp(logits - rowmax)
    masked_e = jnp.where(mask, e, 0.0)
    denom = jnp.sum(masked_e, axis=1, keepdims=True)
    probs_ref[...] = masked_e / denom


@functools.partial(jax.jit, static_argnames=("token_block",))
def _router(h, W, token_block=1024):
    tokens = h.shape[0]
    grid = (tokens // token_block,)
    mask, probs, logits = pl.pallas_call(
        _router_kernel,
        grid=grid,
        in_specs=[
            pl.BlockSpec((token_block, HALF), lambda i: (i, 0)),
            pl.BlockSpec((token_block, HALF), lambda i: (i, 1)),
            pl.BlockSpec((N_EXP, D_MODEL), lambda i: (0, 0)),
        ],
        out_specs=[
            pl.BlockSpec((token_block, N_EXP), lambda i: (i, 0)),
            pl.BlockSpec((token_block, N_EXP), lambda i: (i, 0)),
            pl.BlockSpec((token_block, N_EXP), lambda i: (i, 0)),
        ],
        out_shape=[
            jax.ShapeDtypeStruct((tokens, N_EXP), jnp.bool_),
            jax.ShapeDtypeStruct((tokens, N_EXP), jnp.float32),
            jax.ShapeDtypeStruct((tokens, N_EXP), jnp.float32),
        ],
        compiler_params=pltpu.CompilerParams(
            dimension_semantics=("parallel",),
        ),
    )(h, h, W)
    return mask, probs, logits


def kernel(h, W):
    mask, probs, logits = _router(h, W)
    return (mask, probs, logits, logits)


# speedup vs baseline: 1.5431x; 1.0040x over previous
import functools

import jax
import jax.numpy as jnp
from jax.experimental import pallas as pl
from jax.experimental.pallas import tpu as pltpu

D_MODEL = 4096
N_EXP = 64
TOPK = 8


def _router_kernel(h_ref, w_ref, mask_ref, probs_ref, logits_ref):
    logits = jax.lax.dot_general(
        h_ref[...], w_ref[...],
        dimension_numbers=(((1,), (1,)), ((), ())),
        preferred_element_type=jnp.float32,
    )
    logits_ref[...] = logits

    tb = logits.shape[0]

    work = logits
    mask = jnp.zeros((tb, N_EXP), dtype=jnp.bool_)
    rowmax = None
    for i in range(TOPK):
        m = jnp.max(work, axis=1, keepdims=True)
        if i == 0:
            rowmax = m
        sel = work == m
        mask = jnp.logical_or(mask, sel)
        work = jnp.where(sel, -jnp.inf, work)
    mask_ref[...] = mask

    e = jnp.exp(logits - rowmax)
    masked_e = jnp.where(mask, e, 0.0)
    denom = jnp.sum(masked_e, axis=1, keepdims=True)
    probs_ref[...] = masked_e / denom


@functools.partial(jax.jit, static_argnames=("token_block",))
def _router(h, W, token_block=1024):
    tokens = h.shape[0]
    grid = (tokens // token_block,)
    mask, probs, logits = pl.pallas_call(
        _router_kernel,
        grid=grid,
        in_specs=[
            pl.BlockSpec((token_block, D_MODEL), lambda i: (i, 0)),
            pl.BlockSpec((N_EXP, D_MODEL), lambda i: (0, 0)),
        ],
        out_specs=[
            pl.BlockSpec((token_block, N_EXP), lambda i: (i, 0)),
            pl.BlockSpec((token_block, N_EXP), lambda i: (i, 0)),
            pl.BlockSpec((token_block, N_EXP), lambda i: (i, 0)),
        ],
        out_shape=[
            jax.ShapeDtypeStruct((tokens, N_EXP), jnp.bool_),
            jax.ShapeDtypeStruct((tokens, N_EXP), jnp.float32),
            jax.ShapeDtypeStruct((tokens, N_EXP), jnp.float32),
        ],
        compiler_params=pltpu.CompilerParams(
            dimension_semantics=("parallel",),
        ),
    )(h, W)
    return mask, probs, logits


def kernel(h, W):
    mask, probs, logits = _router(h, W)
    return (mask, probs, logits, logits)
